# baseline (speedup 1.0000x reference)
import jax
import jax.numpy as jnp
from jax.experimental import pallas as pl
from jax.experimental.pallas import tpu as pltpu

_N = 1000
_NP = 1024
_BLK = 256
_UNITS = 64
_C = 256
_EPS = 1e-6
_NEG = -1e30


def _dot(a, b, dims):
    return jax.lax.dot_general(
        a, b, (dims, ((), ())),
        precision=jax.lax.Precision.DEFAULT,
        preferred_element_type=jnp.float32)


def _mm(a, b):
    return _dot(a, b, ((1,), (0,)))


def _mmt(a, b):
    return _dot(a, b, ((1,), (1,)))


def _body(s_row_ref, s_col_ref, boxes_ref, pos_ref, fm_ref,
          W_rank_ref, b_rank_ref, W_feat_ref, b_feat_ref,
          W_geo_ref, b_geo_ref, W_g1_ref, b_g1_ref,
          Wk_ref, bk_ref, Wq_ref, bq_ref, Wv_ref, bv_ref,
          W_emb_ref, b_emb_ref, W_score_ref, b_score_ref,
          out_ref,
          k_s, q_s, v_s, cm_s, corr_s, scol_s, bc_s, br_s):
    i = pl.program_id(0)

    w4 = _mm(W_geo_ref[...], W_g1_ref[...])
    w0 = w4[0:1, 0:1]
    w1 = w4[1:2, 0:1]
    w2 = w4[2:3, 0:1]
    w3 = w4[3:4, 0:1]
    c0 = _mm(b_geo_ref[...], W_g1_ref[...]) + b_g1_ref[...]

    @pl.when(i == 0)
    def _prologue():
        pad_row = jnp.full((1, _NP - _N), _NEG, jnp.float32)
        pad_col = jnp.full((_NP - _N, 1), _NEG, jnp.float32)
        s_row = jnp.concatenate([s_row_ref[...], pad_row], axis=1)
        s_col = jnp.concatenate([s_col_ref[...], pad_col], axis=0)
        scol_s[...] = s_col
        bc = jnp.concatenate(
            [boxes_ref[...], jnp.zeros((_NP - _N, 4), jnp.float32)], axis=0)
        bc_s[...] = bc
        br_s[...] = _mmt(jnp.eye(4, dtype=jnp.float32), bc)
        jr = jax.lax.broadcasted_iota(jnp.int32, (_NP, _NP), 0)
        ic = jax.lax.broadcasted_iota(jnp.int32, (_NP, _NP), 1)
        beats = (s_col > s_row) | ((s_col == s_row) & (jr < ic))
        rank_row = jnp.sum(jnp.where(beats, 1.0, 0.0), axis=0,
                           keepdims=True)
        rank_i = rank_row.astype(jnp.int32)
        PT = jnp.where(rank_i == jr, 1.0, 0.0)
        pos_p = jnp.concatenate(
            [pos_ref[...], jnp.zeros((_NP - _N, _UNITS), jnp.float32)], axis=0)
        rank_emb = _mm(PT, pos_p)
        G = _mm(fm_ref[...], W_feat_ref[...])
        Gp = jnp.concatenate([G, jnp.zeros((_NP - _N, _UNITS), jnp.float32)],
                             axis=0)
        f = (_mm(rank_emb, W_rank_ref[...]) + b_rank_ref[...]
             + Gp + b_feat_ref[...])
        k_s[...] = _mm(f, Wk_ref[...]) + bk_ref[...]
        q_s[...] = _mm(f, Wq_ref[...]) + bq_ref[...]
        v = _mm(f, Wv_ref[...]) + bv_ref[...]
        ones = jnp.ones((_NP, 1), jnp.float32)
        v_s[...] = jnp.concatenate([v, ones], axis=1)
        corr_s[...] = 1e-4 * jnp.sum(v_s[_N:, :], axis=0, keepdims=True)
        br = br_s[...]
        wm = br[2:3, :] - br[0:1, :] + _EPS
        hm = br[3:4, :] - br[1:2, :] + _EPS
        cm = w2 * jnp.log(wm) + w3 * jnp.log(hm) + c0
        lane = jax.lax.broadcasted_iota(jnp.int32, (1, _NP), 1)
        cm_s[...] = jnp.where(lane < _N, cm, _NEG)

    blk = pl.ds(i * _BLK, _BLK)
    bx = bc_s[blk, :]
    xm = bx[:, 0:1]
    ym = bx[:, 1:2]
    xM = bx[:, 2:3]
    yM = bx[:, 3:4]
    wn = xM - xm + _EPS
    hn = yM - ym + _EPS
    cxn = (xm + xM) * 0.5
    cyn = (ym + yM) * 0.5

    cxm = (br_s[0:1, :] + br_s[2:3, :]) * 0.5
    cym = (br_s[1:2, :] + br_s[3:4, :]) * 0.5

    dxl = jnp.log(jnp.abs(cxn - cxm) + _EPS * wn)
    dyl = jnp.log(jnp.abs(cyn - cym) + _EPS * hn)
    row_term = (w0 + w2) * jnp.log(wn) + (w1 + w3) * jnp.log(hn)
    t = w0 * dxl + w1 * dyl + (cm_s[...] - row_term)
    gw = jnp.maximum(t, 0.0)

    inv_sqrt_u = 1.0 / jnp.sqrt(jnp.float32(_UNITS))
    app = _mmt(k_s[blk, :], q_s[...]) * inv_sqrt_u
    wts = jnp.maximum(gw * jnp.exp(app), 1e-4)
    rel_all = _mm(wts, v_s[...]) - corr_s[...]
    num = rel_all[:, :_UNITS]
    den = rel_all[:, _UNITS:]

    wfin = _mm(W_emb_ref[...], W_score_ref[...])
    cfin = _mm(b_emb_ref[...], W_score_ref[...]) + b_score_ref[...]
    logit = _mm(num, wfin) / den + cfin
    out_ref[...] = scol_s[blk, :] * jax.nn.sigmoid(logit)


def kernel(scores, feature_map, boxes, pos_enc, W_rank, b_rank, W_feat, b_feat,
           W_geo, b_geo, W_g1, b_g1, Wk, bk, Wq, bq, Wv, bv, W_emb, b_emb,
           W_score, b_score):
    n = scores.shape[1]
    s_row = scores.reshape(1, n).astype(jnp.float32)
    s_col = scores.reshape(n, 1).astype(jnp.float32)
    bxs = boxes.reshape(n, 4).astype(jnp.float32)
    fm = feature_map.reshape(n, _C).astype(jnp.float32)

    r2 = lambda a: a.reshape(1, -1).astype(jnp.float32)

    full = lambda shape: pl.BlockSpec(shape, lambda i: (0,) * len(shape))
    grid = _NP // _BLK
    out = pl.pallas_call(
        _body,
        grid=(grid,),
        in_specs=[
            full((1, _N)), full((_N, 1)), full((_N, 4)),
            full((_N, _UNITS)), full((_N, _C)),
            full((_UNITS, _UNITS)), full((1, _UNITS)),
            full((_C, _UNITS)), full((1, _UNITS)),
            full((4, _UNITS)), full((1, _UNITS)),
            full((_UNITS, 1)), full((1, 1)),
            full((_UNITS, _UNITS)), full((1, _UNITS)),
            full((_UNITS, _UNITS)), full((1, _UNITS)),
            full((_UNITS, _UNITS)), full((1, _UNITS)),
            full((_UNITS, _UNITS)), full((1, _UNITS)),
            full((_UNITS, 1)), full((1, 1)),
        ],
        out_specs=pl.BlockSpec((_BLK, 1), lambda i: (i, 0)),
        out_shape=jax.ShapeDtypeStruct((_N, 1), jnp.float32),
        scratch_shapes=[
            pltpu.VMEM((_NP, _UNITS), jnp.float32),
            pltpu.VMEM((_NP, _UNITS), jnp.float32),
            pltpu.VMEM((_NP, _UNITS + 1), jnp.float32),
            pltpu.VMEM((1, _NP), jnp.float32),
            pltpu.VMEM((1, _UNITS + 1), jnp.float32),
            pltpu.VMEM((_NP, 1), jnp.float32),
            pltpu.VMEM((_NP, 4), jnp.float32),
            pltpu.VMEM((4, _NP), jnp.float32),
        ],
    )(s_row, s_col, bxs, pos_enc.astype(jnp.float32), fm,
      W_rank.astype(jnp.float32), r2(b_rank),
      W_feat.astype(jnp.float32), r2(b_feat),
      W_geo.astype(jnp.float32), r2(b_geo),
      W_g1.astype(jnp.float32), r2(b_g1),
      Wk.astype(jnp.float32), r2(bk),
      Wq.astype(jnp.float32), r2(bq),
      Wv.astype(jnp.float32), r2(bv),
      W_emb.astype(jnp.float32), r2(b_emb),
      W_score.astype(jnp.float32), r2(b_score))
    return out.reshape(1, n)


# speedup vs baseline: 1.1232x; 1.1232x over previous
import jax
import jax.numpy as jnp
from jax.experimental import pallas as pl

_N = 1000
_NP = 1024
_UNITS = 64
_C = 256
_EPS = 1e-6
_NEG = -1e30


def _dot(a, b, dims):
    return jax.lax.dot_general(
        a, b, (dims, ((), ())),
        precision=jax.lax.Precision.DEFAULT,
        preferred_element_type=jnp.float32)


def _mm(a, b):
    return _dot(a, b, ((1,), (0,)))


def _mmt(a, b):
    return _dot(a, b, ((1,), (1,)))


def _body(s_row_ref, s_col_ref, boxes_ref, pos_ref, fm_ref,
          W_rank_ref, b_rank_ref, W_feat_ref, b_feat_ref,
          W_geo_ref, b_geo_ref, W_g1_ref, b_g1_ref,
          Wk_ref, bk_ref, Wq_ref, bq_ref, Wv_ref, bv_ref,
          W_emb_ref, b_emb_ref, W_score_ref, b_score_ref,
          out_ref):
    w4 = _mm(W_geo_ref[...], W_g1_ref[...])
    w0 = w4[0:1, 0:1]
    w1 = w4[1:2, 0:1]
    w2 = w4[2:3, 0:1]
    w3 = w4[3:4, 0:1]
    c0 = _mm(b_geo_ref[...], W_g1_ref[...]) + b_g1_ref[...]

    pad_row = jnp.full((1, _NP - _N), _NEG, jnp.float32)
    pad_col = jnp.full((_NP - _N, 1), _NEG, jnp.float32)
    s_row = jnp.concatenate([s_row_ref[...], pad_row], axis=1)
    s_col = jnp.concatenate([s_col_ref[...], pad_col], axis=0)
    bc = jnp.concatenate(
        [boxes_ref[...], jnp.zeros((_NP - _N, 4), jnp.float32)], axis=0)
    br = _mmt(jnp.eye(4, dtype=jnp.float32), bc)

    jr = jax.lax.broadcasted_iota(jnp.int32, (_NP, _NP), 0)
    ic = jax.lax.broadcasted_iota(jnp.int32, (_NP, _NP), 1)
    beats = (s_col > s_row) | ((s_col == s_row) & (jr < ic))
    rank_row = jnp.sum(jnp.where(beats, 1.0, 0.0), axis=0,
                       keepdims=True)
    rank_i = rank_row.astype(jnp.int32)
    PT = jnp.where(rank_i == jr, 1.0, 0.0)
    pos_p = jnp.concatenate(
        [pos_ref[...], jnp.zeros((_NP - _N, _UNITS), jnp.float32)], axis=0)
    rank_emb = _mm(PT, pos_p)
    G = _mm(fm_ref[...], W_feat_ref[...])
    Gp = jnp.concatenate([G, jnp.zeros((_NP - _N, _UNITS), jnp.float32)],
                         axis=0)
    f = (_mm(rank_emb, W_rank_ref[...]) + b_rank_ref[...]
         + Gp + b_feat_ref[...])
    k = _mm(f, Wk_ref[...]) + bk_ref[...]
    q = _mm(f, Wq_ref[...]) + bq_ref[...]
    v = _mm(f, Wv_ref[...]) + bv_ref[...]
    ones = jnp.ones((_NP, 1), jnp.float32)
    v65 = jnp.concatenate([v, ones], axis=1)
    corr = 1e-4 * jnp.sum(v65[_N:, :], axis=0, keepdims=True)

    wm = br[2:3, :] - br[0:1, :] + _EPS
    hm = br[3:4, :] - br[1:2, :] + _EPS
    cm = w2 * jnp.log(wm) + w3 * jnp.log(hm) + c0
    lane = jax.lax.broadcasted_iota(jnp.int32, (1, _NP), 1)
    cm = jnp.where(lane < _N, cm, _NEG)

    xm = bc[:, 0:1]
    ym = bc[:, 1:2]
    xM = bc[:, 2:3]
    yM = bc[:, 3:4]
    wn = xM - xm + _EPS
    hn = yM - ym + _EPS
    cxn = (xm + xM) * 0.5
    cyn = (ym + yM) * 0.5
    cxm = (br[0:1, :] + br[2:3, :]) * 0.5
    cym = (br[1:2, :] + br[3:4, :]) * 0.5

    dxl = jnp.log(jnp.abs(cxn - cxm) + _EPS * wn)
    dyl = jnp.log(jnp.abs(cyn - cym) + _EPS * hn)
    row_term = (w0 + w2) * jnp.log(wn) + (w1 + w3) * jnp.log(hn)
    t = w0 * dxl + w1 * dyl + (cm - row_term)
    gw = jnp.maximum(t, 0.0)

    inv_sqrt_u = 1.0 / jnp.sqrt(jnp.float32(_UNITS))
    app = _mmt(k, q) * inv_sqrt_u
    wts = jnp.maximum(gw * jnp.exp(app), 1e-4)
    rel_all = _mm(wts, v65) - corr
    num = rel_all[:, :_UNITS]
    den = rel_all[:, _UNITS:]

    wfin = _mm(W_emb_ref[...], W_score_ref[...])
    cfin = _mm(b_emb_ref[...], W_score_ref[...]) + b_score_ref[...]
    logit = _mm(num, wfin) / den + cfin
    out_ref[...] = (s_col * jax.nn.sigmoid(logit))[:_N, :]


def kernel(scores, feature_map, boxes, pos_enc, W_rank, b_rank, W_feat, b_feat,
           W_geo, b_geo, W_g1, b_g1, Wk, bk, Wq, bq, Wv, bv, W_emb, b_emb,
           W_score, b_score):
    n = scores.shape[1]
    s_row = scores.reshape(1, n).astype(jnp.float32)
    s_col = scores.reshape(n, 1).astype(jnp.float32)
    bxs = boxes.reshape(n, 4).astype(jnp.float32)
    fm = feature_map.reshape(n, _C).astype(jnp.float32)

    r2 = lambda a: a.reshape(1, -1).astype(jnp.float32)

    out = pl.pallas_call(
        _body,
        out_shape=jax.ShapeDtypeStruct((_N, 1), jnp.float32),
    )(s_row, s_col, bxs, pos_enc.astype(jnp.float32), fm,
      W_rank.astype(jnp.float32), r2(b_rank),
      W_feat.astype(jnp.float32), r2(b_feat),
      W_geo.astype(jnp.float32), r2(b_geo),
      W_g1.astype(jnp.float32), r2(b_g1),
      Wk.astype(jnp.float32), r2(bk),
      Wq.astype(jnp.float32), r2(bq),
      Wv.astype(jnp.float32), r2(bv),
      W_emb.astype(jnp.float32), r2(b_emb),
      W_score.astype(jnp.float32), r2(b_score))
    return out.reshape(1, n)
